# baseline (speedup 1.0000x reference)
import functools

import jax
import jax.numpy as jnp
from jax import lax
from jax.experimental import pallas as pl
from jax.experimental.pallas import tpu as pltpu
from jax.experimental.pallas import tpu_sc as plsc

N = 10000
D = 128
NC = 2
NT = 16
CHUNK = 128
IB = 8
P_PAD = 10112
ROWS_P = P_PAD // NT
DEG_PAD = 10240
ROWS_D = DEG_PAD // NT
RB = 400
GRID = N // RB

_sc_mesh = plsc.VectorSubcoreMesh(core_axis_name="c", subcore_axis_name="s")


def _deg_call(dst_flat, e_pad):
    e_w = e_pad // (NC * NT)

    @functools.partial(
        pl.kernel,
        out_type=jax.ShapeDtypeStruct((NC * DEG_PAD,), jnp.float32),
        mesh=_sc_mesh,
        scratch_types=[
            pltpu.VMEM((e_w,), jnp.int32),
            pltpu.VMEM((DEG_PAD,), jnp.float32),
            pltpu.VMEM((ROWS_D,), jnp.float32),
            pltpu.VMEM((ROWS_D,), jnp.float32),
            pltpu.VMEM_SHARED((NT, DEG_PAD), jnp.float32),
        ],
        compiler_params=pltpu.CompilerParams(needs_layout_passes=False),
    )
    def deg_kernel(dst_hbm, out_hbm, dstv, hist, buf, acc, shared):
        c = lax.axis_index("c")
        t = lax.axis_index("s")
        pltpu.sync_copy(dst_hbm.at[pl.ds((c * NT + t) * e_w, e_w)], dstv)
        zero16 = jnp.zeros((16,), jnp.float32)
        ones16 = jnp.ones((16,), jnp.float32)

        def zbody(i, carry):
            hist[pl.ds(i * 16, 16)] = zero16
            return carry

        lax.fori_loop(0, DEG_PAD // 16, zbody, 0)

        def scat(i, carry):
            idx = dstv[pl.ds(i * 16, 16)]
            plsc.addupdate_scatter(hist, [idx], ones16)
            return carry

        lax.fori_loop(0, e_w // 16, scat, 0)
        pltpu.sync_copy(hist, shared.at[t])
        plsc.subcore_barrier()

        def z2(i, carry):
            acc[pl.ds(i * 16, 16)] = zero16
            return carry

        lax.fori_loop(0, ROWS_D // 16, z2, 0)
        for r in range(NT):
            pltpu.sync_copy(shared.at[r, pl.ds(t * ROWS_D, ROWS_D)], buf)

            def addb(i, carry):
                acc[pl.ds(i * 16, 16)] = acc[pl.ds(i * 16, 16)] + buf[pl.ds(i * 16, 16)]
                return carry

            lax.fori_loop(0, ROWS_D // 16, addb, 0)
        pltpu.sync_copy(acc, out_hbm.at[pl.ds(c * DEG_PAD + t * ROWS_D, ROWS_D)])

    return deg_kernel(dst_flat)


def _propagate(gtab, src_idx, dst_idx, zeros_blk, c_w):
    nb = c_w // IB

    @functools.partial(
        pl.kernel,
        out_type=jax.ShapeDtypeStruct((NC * P_PAD, D), jnp.float32),
        mesh=_sc_mesh,
        scratch_types=[
            pltpu.VMEM((c_w, CHUNK), jnp.int32),
            pltpu.VMEM((IB, CHUNK), jnp.int32),
            pltpu.VMEM((IB, CHUNK), jnp.int32),
            pltpu.VMEM((CHUNK, D), jnp.float32),
            pltpu.VMEM((CHUNK, D), jnp.float32),
            pltpu.VMEM_SHARED((P_PAD, D), jnp.float32),
            pltpu.SemaphoreType.DMA,
            pltpu.SemaphoreType.DMA,
            pltpu.SemaphoreType.DMA,
        ],
        compiler_params=pltpu.CompilerParams(needs_layout_passes=False),
    )
    def prop_kernel(gtab_hbm, src_hbm, dst_hbm, z_hbm, out_hbm,
                    src_v, dst_a, dst_b, rows0, rows1, accum, sg0, sg1, sdi):
        c = lax.axis_index("c")
        t = lax.axis_index("s")
        base = t * ROWS_P
        wbase = (c * NT + t) * c_w
        pltpu.sync_copy(z_hbm, accum.at[pl.ds(base, ROWS_P)])
        pltpu.sync_copy(src_hbm.at[pl.ds(wbase, c_w)], src_v)
        pltpu.sync_copy(dst_hbm.at[pl.ds(wbase, IB)], dst_a)
        plsc.subcore_barrier()

        pltpu.async_copy(gtab_hbm.at[src_v.at[0]], rows0, sg0)
        pltpu.async_copy(gtab_hbm.at[src_v.at[1]], rows1, sg1)

        def inner(kb, cur, nxt):
            @pl.when(kb + 1 < nb)
            def _():
                pltpu.async_copy(
                    dst_hbm.at[pl.ds(wbase + (kb + 1) * IB, IB)], nxt, sdi)

            for q in range(IB):
                j = kb * IB + q
                rp, sp = (rows0, sg0) if q % 2 == 0 else (rows1, sg1)
                pltpu.make_async_copy(gtab_hbm.at[src_v.at[j]], rp, sp).wait()
                pltpu.sync_copy(rp, accum.at[cur.at[q]], add=True)

                @pl.when(j + 2 < c_w)
                def _():
                    pltpu.async_copy(gtab_hbm.at[src_v.at[j + 2]], rp, sp)

            @pl.when(kb + 1 < nb)
            def _():
                pltpu.make_async_copy(
                    dst_hbm.at[pl.ds(wbase + (kb + 1) * IB, IB)], nxt, sdi).wait()

        def step(kb, carry):
            @pl.when(kb % 2 == 0)
            def _():
                inner(kb, dst_a, dst_b)

            @pl.when(kb % 2 == 1)
            def _():
                inner(kb, dst_b, dst_a)

            return carry

        lax.fori_loop(0, nb, step, 0)
        plsc.subcore_barrier()
        pltpu.sync_copy(accum.at[pl.ds(base, ROWS_P)],
                        out_hbm.at[pl.ds(c * P_PAD + base, ROWS_P)])

    return prop_kernel(gtab, src_idx, dst_idx, zeros_blk)


def _tc_first(x, W, deg0, deg1):
    def body(x_ref, w_ref, d0_ref, d1_ref, g_ref, dinv_ref):
        dinv = lax.rsqrt(jnp.maximum(d0_ref[...] + d1_ref[...], 1.0))
        m = jnp.dot(x_ref[...], w_ref[...], preferred_element_type=jnp.float32)
        g_ref[...] = m * dinv
        dinv_ref[...] = dinv

    return pl.pallas_call(
        body,
        grid=(GRID,),
        in_specs=[
            pl.BlockSpec((RB, D), lambda i: (i, 0)),
            pl.BlockSpec((D, D), lambda i: (0, 0)),
            pl.BlockSpec((RB, 1), lambda i: (i, 0)),
            pl.BlockSpec((RB, 1), lambda i: (i, 0)),
        ],
        out_specs=[
            pl.BlockSpec((RB, D), lambda i: (i, 0)),
            pl.BlockSpec((RB, 1), lambda i: (i, 0)),
        ],
        out_shape=[
            jax.ShapeDtypeStruct((N, D), jnp.float32),
            jax.ShapeDtypeStruct((N, 1), jnp.float32),
        ],
    )(x, W, deg0, deg1)


def _tc_mid(s, dinv, b, res, W):
    has_res = res is not None

    def body(*refs):
        if has_res:
            s_ref, dinv_ref, b_ref, res_ref, w_ref, h_ref, g_ref = refs
        else:
            s_ref, dinv_ref, b_ref, w_ref, h_ref, g_ref = refs
        dv = dinv_ref[...]
        h = (s_ref[0] + s_ref[1]) * dv + b_ref[...]
        if has_res:
            h = h + res_ref[...]
        h = jnp.maximum(h, 0.0)
        h_ref[...] = h
        g_ref[...] = jnp.dot(h, w_ref[...], preferred_element_type=jnp.float32) * dv

    in_specs = [
        pl.BlockSpec((NC, RB, D), lambda i: (0, i, 0)),
        pl.BlockSpec((RB, 1), lambda i: (i, 0)),
        pl.BlockSpec((1, D), lambda i: (0, 0)),
    ]
    args = [s, dinv, b]
    if has_res:
        in_specs.append(pl.BlockSpec((RB, D), lambda i: (i, 0)))
        args.append(res)
    in_specs.append(pl.BlockSpec((D, D), lambda i: (0, 0)))
    args.append(W)
    return pl.pallas_call(
        body,
        grid=(GRID,),
        in_specs=in_specs,
        out_specs=[
            pl.BlockSpec((RB, D), lambda i: (i, 0)),
            pl.BlockSpec((RB, D), lambda i: (i, 0)),
        ],
        out_shape=[
            jax.ShapeDtypeStruct((N, D), jnp.float32),
            jax.ShapeDtypeStruct((N, D), jnp.float32),
        ],
    )(*args)


def _tc_last(s, dinv, b):
    def body(s_ref, dinv_ref, b_ref, out_ref):
        out_ref[...] = (s_ref[0] + s_ref[1]) * dinv_ref[...] + b_ref[...]

    return pl.pallas_call(
        body,
        grid=(GRID,),
        in_specs=[
            pl.BlockSpec((NC, RB, D), lambda i: (0, i, 0)),
            pl.BlockSpec((RB, 1), lambda i: (i, 0)),
            pl.BlockSpec((1, D), lambda i: (0, 0)),
        ],
        out_specs=pl.BlockSpec((RB, D), lambda i: (i, 0)),
        out_shape=jax.ShapeDtypeStruct((N, D), jnp.float32),
    )(s, dinv, b)


def kernel(x, edge_index, W_in, b_in, W_h0, b_h0, W_h1, b_h1, W_out, b_out):
    src = edge_index[0]
    dst = edge_index[1]
    e_tot = src.shape[0] + N
    c_w = -(-e_tot // (NC * NT * CHUNK))
    c_w = -(-c_w // IB) * IB
    e_pad = NC * NT * c_w * CHUNK
    loop_idx = jnp.arange(N, dtype=jnp.int32)
    pad = e_pad - e_tot
    src_f = jnp.concatenate([src, loop_idx, jnp.zeros((pad,), jnp.int32)])
    dst_f = jnp.concatenate([dst, loop_idx, jnp.full((pad,), N, jnp.int32)])
    src_idx = src_f.reshape(NC * NT * c_w, CHUNK)
    dst_idx = dst_f.reshape(NC * NT * c_w, CHUNK)
    zeros_blk = jnp.zeros((ROWS_P, D), jnp.float32)

    deg = _deg_call(dst_f, e_pad).reshape(NC, DEG_PAD, 1)
    g1, dinv = _tc_first(x, W_in, deg[0], deg[1])
    s1 = _propagate(g1, src_idx, dst_idx, zeros_blk, c_w)
    h1, g2 = _tc_mid(s1.reshape(NC, P_PAD, D), dinv, b_in.reshape(1, D), None, W_h0)
    s2 = _propagate(g2, src_idx, dst_idx, zeros_blk, c_w)
    h2, g3 = _tc_mid(s2.reshape(NC, P_PAD, D), dinv, b_h0.reshape(1, D), h1, W_h1)
    s3 = _propagate(g3, src_idx, dst_idx, zeros_blk, c_w)
    _, g4 = _tc_mid(s3.reshape(NC, P_PAD, D), dinv, b_h1.reshape(1, D), h2, W_out)
    s4 = _propagate(g4, src_idx, dst_idx, zeros_blk, c_w)
    return _tc_last(s4.reshape(NC, P_PAD, D), dinv, b_out.reshape(1, D))


# speedup vs baseline: 1.0277x; 1.0277x over previous
import functools

import jax
import jax.numpy as jnp
from jax import lax
from jax.experimental import pallas as pl
from jax.experimental.pallas import tpu as pltpu
from jax.experimental.pallas import tpu_sc as plsc

N = 10000
D = 128
NC = 2
NT = 16
CHUNK = 128
IB = 8
P_PAD = 10112
ROWS_P = P_PAD // NT
DEG_PAD = 10240
ROWS_D = DEG_PAD // NT
RB = 400
GRID = N // RB

_sc_mesh = plsc.VectorSubcoreMesh(core_axis_name="c", subcore_axis_name="s")


def _deg_call(dst_flat, e_pad):
    e_w = e_pad // (NC * NT)

    @functools.partial(
        pl.kernel,
        out_type=jax.ShapeDtypeStruct((NC * DEG_PAD,), jnp.float32),
        mesh=_sc_mesh,
        scratch_types=[
            pltpu.VMEM((e_w,), jnp.int32),
            pltpu.VMEM((DEG_PAD,), jnp.float32),
            pltpu.VMEM((ROWS_D,), jnp.float32),
            pltpu.VMEM((ROWS_D,), jnp.float32),
            pltpu.VMEM_SHARED((NT, DEG_PAD), jnp.float32),
        ],
        compiler_params=pltpu.CompilerParams(needs_layout_passes=False),
    )
    def deg_kernel(dst_hbm, out_hbm, dstv, hist, buf, acc, shared):
        c = lax.axis_index("c")
        t = lax.axis_index("s")
        pltpu.sync_copy(dst_hbm.at[pl.ds((c * NT + t) * e_w, e_w)], dstv)
        zero16 = jnp.zeros((16,), jnp.float32)
        ones16 = jnp.ones((16,), jnp.float32)

        def zbody(i, carry):
            hist[pl.ds(i * 16, 16)] = zero16
            return carry

        lax.fori_loop(0, DEG_PAD // 16, zbody, 0)

        def scat(i, carry):
            idx = dstv[pl.ds(i * 16, 16)]
            plsc.addupdate_scatter(hist, [idx], ones16)
            return carry

        lax.fori_loop(0, e_w // 16, scat, 0)
        pltpu.sync_copy(hist, shared.at[t])
        plsc.subcore_barrier()

        def z2(i, carry):
            acc[pl.ds(i * 16, 16)] = zero16
            return carry

        lax.fori_loop(0, ROWS_D // 16, z2, 0)
        for r in range(NT):
            pltpu.sync_copy(shared.at[r, pl.ds(t * ROWS_D, ROWS_D)], buf)

            def addb(i, carry):
                acc[pl.ds(i * 16, 16)] = acc[pl.ds(i * 16, 16)] + buf[pl.ds(i * 16, 16)]
                return carry

            lax.fori_loop(0, ROWS_D // 16, addb, 0)
        pltpu.sync_copy(acc, out_hbm.at[pl.ds(c * DEG_PAD + t * ROWS_D, ROWS_D)])

    return deg_kernel(dst_flat)


def _propagate(gtab, src_idx, dst_idx, zeros_blk, c_w):
    nb = c_w // IB

    @functools.partial(
        pl.kernel,
        out_type=jax.ShapeDtypeStruct((NC * P_PAD, D), jnp.float32),
        mesh=_sc_mesh,
        scratch_types=[
            pltpu.VMEM((c_w, CHUNK), jnp.int32),
            pltpu.VMEM((IB, CHUNK), jnp.int32),
            pltpu.VMEM((IB, CHUNK), jnp.int32),
            pltpu.VMEM((CHUNK, D), jnp.float32),
            pltpu.VMEM((CHUNK, D), jnp.float32),
            pltpu.VMEM_SHARED((P_PAD, D), jnp.float32),
            pltpu.SemaphoreType.DMA,
            pltpu.SemaphoreType.DMA,
            pltpu.SemaphoreType.DMA,
            pltpu.SemaphoreType.DMA,
        ],
        compiler_params=pltpu.CompilerParams(needs_layout_passes=False),
    )
    def prop_kernel(gtab_hbm, src_hbm, dst_hbm, z_hbm, out_hbm,
                    src_v, dst_a, dst_b, rows0, rows1, accum, sg0, sg1, sdi, ss):
        c = lax.axis_index("c")
        t = lax.axis_index("s")
        base = t * ROWS_P
        wbase = (c * NT + t) * c_w
        pltpu.sync_copy(z_hbm, accum.at[pl.ds(base, ROWS_P)])
        pltpu.sync_copy(src_hbm.at[pl.ds(wbase, c_w)], src_v)
        pltpu.sync_copy(dst_hbm.at[pl.ds(wbase, IB)], dst_a)
        plsc.subcore_barrier()

        pltpu.async_copy(gtab_hbm.at[src_v.at[0]], rows0, sg0)
        pltpu.async_copy(gtab_hbm.at[src_v.at[1]], rows1, sg1)

        def inner(kb, cur, nxt):
            @pl.when(kb + 1 < nb)
            def _():
                pltpu.async_copy(
                    dst_hbm.at[pl.ds(wbase + (kb + 1) * IB, IB)], nxt, sdi)

            for qp in range(IB // 2):
                q0, q1 = 2 * qp, 2 * qp + 1
                j0 = kb * IB + q0
                j1 = kb * IB + q1
                pltpu.make_async_copy(gtab_hbm.at[src_v.at[j0]], rows0, sg0).wait()
                pltpu.make_async_copy(gtab_hbm.at[src_v.at[j1]], rows1, sg1).wait()
                pltpu.async_copy(rows0, accum.at[cur.at[q0]], ss, add=True)
                pltpu.async_copy(rows1, accum.at[cur.at[q1]], ss, add=True)
                pltpu.make_async_copy(rows0, accum.at[cur.at[q0]], ss).wait()
                pltpu.make_async_copy(rows1, accum.at[cur.at[q1]], ss).wait()

                @pl.when(j0 + 2 < c_w)
                def _():
                    pltpu.async_copy(gtab_hbm.at[src_v.at[j0 + 2]], rows0, sg0)

                @pl.when(j1 + 2 < c_w)
                def _():
                    pltpu.async_copy(gtab_hbm.at[src_v.at[j1 + 2]], rows1, sg1)

            @pl.when(kb + 1 < nb)
            def _():
                pltpu.make_async_copy(
                    dst_hbm.at[pl.ds(wbase + (kb + 1) * IB, IB)], nxt, sdi).wait()

        def step(kb, carry):
            @pl.when(kb % 2 == 0)
            def _():
                inner(kb, dst_a, dst_b)

            @pl.when(kb % 2 == 1)
            def _():
                inner(kb, dst_b, dst_a)

            return carry

        lax.fori_loop(0, nb, step, 0)
        plsc.subcore_barrier()
        pltpu.sync_copy(accum.at[pl.ds(base, ROWS_P)],
                        out_hbm.at[pl.ds(c * P_PAD + base, ROWS_P)])

    return prop_kernel(gtab, src_idx, dst_idx, zeros_blk)


def _tc_first(x, W, deg0, deg1):
    def body(x_ref, w_ref, d0_ref, d1_ref, g_ref, dinv_ref):
        dinv = lax.rsqrt(jnp.maximum(d0_ref[...] + d1_ref[...], 1.0))
        m = jnp.dot(x_ref[...], w_ref[...], preferred_element_type=jnp.float32)
        g_ref[...] = m * dinv
        dinv_ref[...] = dinv

    return pl.pallas_call(
        body,
        grid=(GRID,),
        in_specs=[
            pl.BlockSpec((RB, D), lambda i: (i, 0)),
            pl.BlockSpec((D, D), lambda i: (0, 0)),
            pl.BlockSpec((RB, 1), lambda i: (i, 0)),
            pl.BlockSpec((RB, 1), lambda i: (i, 0)),
        ],
        out_specs=[
            pl.BlockSpec((RB, D), lambda i: (i, 0)),
            pl.BlockSpec((RB, 1), lambda i: (i, 0)),
        ],
        out_shape=[
            jax.ShapeDtypeStruct((N, D), jnp.float32),
            jax.ShapeDtypeStruct((N, 1), jnp.float32),
        ],
    )(x, W, deg0, deg1)


def _tc_mid(s, dinv, b, res, W):
    has_res = res is not None

    def body(*refs):
        if has_res:
            s_ref, dinv_ref, b_ref, res_ref, w_ref, h_ref, g_ref = refs
        else:
            s_ref, dinv_ref, b_ref, w_ref, h_ref, g_ref = refs
        dv = dinv_ref[...]
        h = (s_ref[0] + s_ref[1]) * dv + b_ref[...]
        if has_res:
            h = h + res_ref[...]
        h = jnp.maximum(h, 0.0)
        h_ref[...] = h
        g_ref[...] = jnp.dot(h, w_ref[...], preferred_element_type=jnp.float32) * dv

    in_specs = [
        pl.BlockSpec((NC, RB, D), lambda i: (0, i, 0)),
        pl.BlockSpec((RB, 1), lambda i: (i, 0)),
        pl.BlockSpec((1, D), lambda i: (0, 0)),
    ]
    args = [s, dinv, b]
    if has_res:
        in_specs.append(pl.BlockSpec((RB, D), lambda i: (i, 0)))
        args.append(res)
    in_specs.append(pl.BlockSpec((D, D), lambda i: (0, 0)))
    args.append(W)
    return pl.pallas_call(
        body,
        grid=(GRID,),
        in_specs=in_specs,
        out_specs=[
            pl.BlockSpec((RB, D), lambda i: (i, 0)),
            pl.BlockSpec((RB, D), lambda i: (i, 0)),
        ],
        out_shape=[
            jax.ShapeDtypeStruct((N, D), jnp.float32),
            jax.ShapeDtypeStruct((N, D), jnp.float32),
        ],
    )(*args)


def _tc_last(s, dinv, b):
    def body(s_ref, dinv_ref, b_ref, out_ref):
        out_ref[...] = (s_ref[0] + s_ref[1]) * dinv_ref[...] + b_ref[...]

    return pl.pallas_call(
        body,
        grid=(GRID,),
        in_specs=[
            pl.BlockSpec((NC, RB, D), lambda i: (0, i, 0)),
            pl.BlockSpec((RB, 1), lambda i: (i, 0)),
            pl.BlockSpec((1, D), lambda i: (0, 0)),
        ],
        out_specs=pl.BlockSpec((RB, D), lambda i: (i, 0)),
        out_shape=jax.ShapeDtypeStruct((N, D), jnp.float32),
    )(s, dinv, b)


def kernel(x, edge_index, W_in, b_in, W_h0, b_h0, W_h1, b_h1, W_out, b_out):
    src = edge_index[0]
    dst = edge_index[1]
    e_tot = src.shape[0] + N
    c_w = -(-e_tot // (NC * NT * CHUNK))
    c_w = -(-c_w // IB) * IB
    e_pad = NC * NT * c_w * CHUNK
    loop_idx = jnp.arange(N, dtype=jnp.int32)
    pad = e_pad - e_tot
    src_f = jnp.concatenate([src, loop_idx, jnp.zeros((pad,), jnp.int32)])
    dst_f = jnp.concatenate([dst, loop_idx, jnp.full((pad,), N, jnp.int32)])
    src_idx = src_f.reshape(NC * NT * c_w, CHUNK)
    dst_idx = dst_f.reshape(NC * NT * c_w, CHUNK)
    zeros_blk = jnp.zeros((ROWS_P, D), jnp.float32)

    deg = _deg_call(dst_f, e_pad).reshape(NC, DEG_PAD, 1)
    g1, dinv = _tc_first(x, W_in, deg[0], deg[1])
    s1 = _propagate(g1, src_idx, dst_idx, zeros_blk, c_w)
    h1, g2 = _tc_mid(s1.reshape(NC, P_PAD, D), dinv, b_in.reshape(1, D), None, W_h0)
    s2 = _propagate(g2, src_idx, dst_idx, zeros_blk, c_w)
    h2, g3 = _tc_mid(s2.reshape(NC, P_PAD, D), dinv, b_h0.reshape(1, D), h1, W_h1)
    s3 = _propagate(g3, src_idx, dst_idx, zeros_blk, c_w)
    _, g4 = _tc_mid(s3.reshape(NC, P_PAD, D), dinv, b_h1.reshape(1, D), h2, W_out)
    s4 = _propagate(g4, src_idx, dst_idx, zeros_blk, c_w)
    return _tc_last(s4.reshape(NC, P_PAD, D), dinv, b_out.reshape(1, D))
